# baseline (speedup 1.0000x reference)
import functools

import jax
import jax.numpy as jnp
from jax import lax
from jax.experimental import pallas as pl
from jax.experimental.pallas import tpu as pltpu
from jax.experimental.pallas import tpu_sc as plsc

NC = 2
NS = 16
NW = NC * NS
LANES = 16
CHUNK = 128
NBUF = 4


def _make_lookup(n_chunks, v, d):
  cpw = n_chunks // NW
  t_steps = cpw // NBUF
  mesh = plsc.VectorSubcoreMesh(
      core_axis_name="c", subcore_axis_name="s", num_cores=NC, num_subcores=NS
  )

  @functools.partial(
      pl.kernel,
      out_type=jax.ShapeDtypeStruct((n_chunks * CHUNK, d), jnp.float32),
      mesh=mesh,
      scratch_types=dict(
          idx_v=pltpu.VMEM((cpw, CHUNK), jnp.int32),
          tti_v=pltpu.VMEM((cpw, CHUNK), jnp.int32),
          tts=pltpu.VMEM_SHARED((2, d), jnp.float32),
          sbuf=pltpu.VMEM((NBUF, CHUNK, d), jnp.float32),
          psems=[pltpu.SemaphoreType.DMA] * NBUF,
          gsems=[pltpu.SemaphoreType.DMA] * NBUF,
          ssems=[pltpu.SemaphoreType.DMA] * NBUF,
      ),
  )
  def lookup(idx_hbm, tti_hbm, wtab_hbm, ttab_hbm, out_hbm,
             idx_v, tti_v, tts, sbuf, psems, gsems, ssems):
    wid = lax.axis_index("s") * NC + lax.axis_index("c")
    c0 = wid * cpw

    pltpu.sync_copy(idx_hbm.at[pl.ds(c0, cpw)], idx_v)
    pltpu.sync_copy(tti_hbm.at[pl.ds(c0, cpw)], tti_v)

    @pl.when(lax.axis_index("s") == 0)
    def _():
      pltpu.sync_copy(ttab_hbm, tts)

    plsc.subcore_barrier()

    def prefill(g, b):
      pltpu.async_copy(tts.at[tti_v.at[g]], sbuf.at[b], psems[b])

    def prefill_wait(g, b):
      pltpu.make_async_copy(tts.at[tti_v.at[g]], sbuf.at[b], psems[b]).wait()

    def gather_add(g, b):
      pltpu.async_copy(wtab_hbm.at[idx_v.at[g]], sbuf.at[b], gsems[b], add=True)

    def gather_wait(g, b):
      pltpu.make_async_copy(wtab_hbm.at[idx_v.at[g]], sbuf.at[b], gsems[b]).wait()

    def scatter(g, b):
      pltpu.async_copy(
          sbuf.at[b], out_hbm.at[pl.ds((c0 + g) * CHUNK, CHUNK)], ssems[b]
      )

    def scatter_wait(g, b):
      pltpu.make_async_copy(
          sbuf.at[b], out_hbm.at[pl.ds((c0 + g) * CHUNK, CHUNK)], ssems[b]
      ).wait()

    def outer(t, carry):
      for b in range(NBUF):
        g = t * NBUF + b
        @pl.when(t > 0)
        def _():
          scatter_wait(g - NBUF, b)
          prefill(g, b)

        @pl.when(t == 0)
        def _():
          prefill(g, b)

        bh = (b - 1) % NBUF
        if b == 0:
          @pl.when(t > 0)
          def _():
            prefill_wait(g - 1, bh)
            gather_add(g - 1, bh)
        else:
          prefill_wait(g - 1, bh)
          gather_add(g - 1, bh)

        bk = (b - 2) % NBUF
        if b <= 1:
          @pl.when(t > 0)
          def _():
            gather_wait(g - 2, bk)
            scatter(g - 2, bk)
        else:
          gather_wait(g - 2, bk)
          scatter(g - 2, bk)
      return carry

    lax.fori_loop(0, t_steps, outer, 0)

    last = cpw - 1
    prefill_wait(last, last % NBUF)
    gather_add(last, last % NBUF)
    gather_wait(last - 1, (last - 1) % NBUF)
    scatter(last - 1, (last - 1) % NBUF)
    gather_wait(last, last % NBUF)
    scatter(last, last % NBUF)
    for b in range(NBUF):
      g = cpw - NBUF + b
      scatter_wait(g, g % NBUF)

  return lookup


def kernel(input_ids, token_type_ids, word_embeddings, token_type_embeddings):
  b, l = input_ids.shape
  v, d = word_embeddings.shape
  n = b * l
  n_chunks = n // CHUNK
  idx2d = input_ids.reshape(n_chunks, CHUNK).astype(jnp.int32)
  tti2d = token_type_ids.reshape(n_chunks, CHUNK).astype(jnp.int32)
  out = _make_lookup(n_chunks, v, d)(
      idx2d, tti2d, word_embeddings, token_type_embeddings.astype(jnp.float32)
  )
  return out.reshape(b, l, d)
